# Initial kernel scaffold; baseline (speedup 1.0000x reference)
import jax
import jax.numpy as jnp
from jax.experimental import pallas as pl


def kernel(x, edge_index, W1, b1, W2, b2):
    raise NotImplementedError("write your pallas kernel here")


# speedup vs baseline: 10.3695x; 10.3695x over previous
import functools

import jax
import jax.numpy as jnp
from jax import lax
from jax.experimental import pallas as pl
from jax.experimental.pallas import tpu as pltpu
from jax.experimental.pallas import tpu_sc as plsc

N = 10000
NP = 10240
E = 320000
F = 128
HID = 128
CLS = 64
NC = 2
NS = 16
K = 125
TCH = E // (NS * K)
DCH = E // (NC * NS * K)
RPT = NP // NS
RB = 2048
GRID = NP // RB

f32 = jnp.float32

_mesh = plsc.VectorSubcoreMesh(
    core_axis_name="c", subcore_axis_name="s", num_cores=NC, num_subcores=NS)

_sc_params = pltpu.CompilerParams(use_tc_tiling_on_sc=False)


def _degree_pass(src3, dst3, zn):
  @functools.partial(
      pl.kernel, mesh=_mesh, compiler_params=_sc_params,
      out_type=(jax.ShapeDtypeStruct((NC, NP), f32),
                jax.ShapeDtypeStruct((NC, NP), f32)),
      scratch_types=[
          pltpu.VMEM_SHARED((NP,), f32),
          pltpu.VMEM_SHARED((NP,), f32),
          pltpu.VMEM((DCH, K), jnp.int32),
          pltpu.VMEM((DCH, K), jnp.int32),
          pltpu.VMEM((128,), f32),
      ])
  def body(src_h, dst_h, zn_h, do_h, di_h, do_sh, di_sh, sidx, didx, ones_v):
    c = lax.axis_index("c")
    s = lax.axis_index("s")
    w = c * NS + s
    pltpu.sync_copy(src_h.at[w // 2, pl.ds((w % 2) * DCH, DCH)], sidx)
    pltpu.sync_copy(dst_h.at[w // 2, pl.ds((w % 2) * DCH, DCH)], didx)

    @pl.when(s == 0)
    def _():
      pltpu.sync_copy(zn_h, do_sh)
      pltpu.sync_copy(zn_h, di_sh)

    for j in range(8):
      ones_v[pl.ds(j * 16, 16)] = jnp.ones((16,), f32)
    plsc.subcore_barrier()

    @pl.loop(0, DCH)
    def _(j):
      pltpu.sync_copy(ones_v.at[pl.ds(0, K)], do_sh.at[sidx.at[j]], add=True)
      pltpu.sync_copy(ones_v.at[pl.ds(0, K)], di_sh.at[didx.at[j]], add=True)

    plsc.subcore_barrier()
    pltpu.sync_copy(do_sh.at[pl.ds(s * RPT, RPT)], do_h.at[c, pl.ds(s * RPT, RPT)])
    pltpu.sync_copy(di_sh.at[pl.ds(s * RPT, RPT)], di_h.at[c, pl.ds(s * RPT, RPT)])

  return body(src3, dst3, zn)


def _message_pass(xw, src3, dst3, zeros, dh):
  hch = TCH // 2

  @functools.partial(
      pl.kernel, mesh=_mesh, compiler_params=_sc_params,
      out_type=jax.ShapeDtypeStruct((NC, NP, dh), f32),
      scratch_types=[
          pltpu.VMEM_SHARED((NP, dh), f32),
          pltpu.VMEM_SHARED((NP, dh), f32),
          pltpu.VMEM((hch, K), jnp.int32),
          pltpu.VMEM((hch, K), jnp.int32),
          pltpu.VMEM((K, dh), f32),
          pltpu.VMEM((K, dh), f32),
          pltpu.SemaphoreType.DMA,
          pltpu.SemaphoreType.DMA,
      ])
  def body(xw_h, src_h, dst_h, z_h, out_h, tab, acc, sidx, didx, r0, r1,
           sm0, sm1):
    c = lax.axis_index("c")
    s = lax.axis_index("s")
    pltpu.sync_copy(xw_h.at[c, pl.ds(s * RPT, RPT)], tab.at[pl.ds(s * RPT, RPT)])
    pltpu.sync_copy(z_h.at[pl.ds(s * RPT, RPT)], acc.at[pl.ds(s * RPT, RPT)])
    plsc.subcore_barrier()

    for ph in range(2):
      pltpu.sync_copy(src_h.at[s, pl.ds(ph * hch, hch)], sidx)
      pltpu.sync_copy(dst_h.at[s, pl.ds(ph * hch, hch)], didx)
      pltpu.async_copy(tab.at[sidx.at[0]], r0, sm0)

      @pl.loop(0, hch, step=2)
      def _(g):
        pltpu.async_copy(tab.at[sidx.at[g + 1]], r1, sm1)
        pltpu.make_async_copy(tab.at[sidx.at[g]], r0, sm0).wait()
        pltpu.sync_copy(r0, acc.at[didx.at[g]], add=True)

        @pl.when(g + 2 < hch)
        def _():
          pltpu.async_copy(tab.at[sidx.at[g + 2]], r0, sm0)

        pltpu.make_async_copy(tab.at[sidx.at[g + 1]], r1, sm1).wait()
        pltpu.sync_copy(r1, acc.at[didx.at[g + 1]], add=True)

    plsc.subcore_barrier()
    pltpu.sync_copy(acc.at[pl.ds(s * RPT, RPT)], out_h.at[c, pl.ds(s * RPT, RPT)])

  return body(xw, src3, dst3, zeros)


def _norm(dref):
  deg = dref[0] + dref[1]
  return lax.rsqrt(jnp.maximum(deg, 1.0))


def _m1(x_p, W1, dop):
  def body(x_ref, w_ref, d_ref, o_ref):
    xw = jnp.dot(x_ref[...], w_ref[...], preferred_element_type=f32)
    res = xw * _norm(d_ref)
    o_ref[0] = res[:, :CLS]
    o_ref[1] = res[:, CLS:]

  return pl.pallas_call(
      body, grid=(GRID,),
      in_specs=[
          pl.BlockSpec((RB, F), lambda i: (i, 0)),
          pl.BlockSpec((F, HID), lambda i: (0, 0)),
          pl.BlockSpec((NC, RB, 1), lambda i: (0, i, 0)),
      ],
      out_specs=pl.BlockSpec((NC, RB, CLS), lambda i: (0, i, 0)),
      out_shape=jax.ShapeDtypeStruct((NC, NP, CLS), f32))(x_p, W1, dop)


def _m2(agg1, dop, dip, b1, W2):
  def body(a_ref, do_ref, di_ref, b_ref, w_ref, o_ref):
    agg = jnp.concatenate([a_ref[0], a_ref[1]], axis=1)
    h = jnp.maximum(agg * _norm(di_ref) + b_ref[...], 0.0)
    h = h * _norm(do_ref)
    res = jnp.dot(h, w_ref[...], preferred_element_type=f32)
    o_ref[0] = res[:, :CLS // 2]
    o_ref[1] = res[:, CLS // 2:]

  return pl.pallas_call(
      body, grid=(GRID,),
      in_specs=[
          pl.BlockSpec((NC, RB, CLS), lambda i: (0, i, 0)),
          pl.BlockSpec((NC, RB, 1), lambda i: (0, i, 0)),
          pl.BlockSpec((NC, RB, 1), lambda i: (0, i, 0)),
          pl.BlockSpec((1, HID), lambda i: (0, 0)),
          pl.BlockSpec((HID, CLS), lambda i: (0, 0)),
      ],
      out_specs=pl.BlockSpec((NC, RB, CLS // 2), lambda i: (0, i, 0)),
      out_shape=jax.ShapeDtypeStruct((NC, NP, CLS // 2), f32))(
          agg1, dop, dip, b1, W2)


def _m3(agg2, dip, b2):
  def body(a_ref, di_ref, b_ref, o_ref):
    z = jnp.concatenate([a_ref[0], a_ref[1]], axis=1)
    z = z * _norm(di_ref) + b_ref[...]
    z = z - jnp.max(z, axis=1, keepdims=True)
    ez = jnp.exp(z)
    o_ref[...] = ez / jnp.sum(ez, axis=1, keepdims=True)

  return pl.pallas_call(
      body, grid=(GRID,),
      in_specs=[
          pl.BlockSpec((NC, RB, CLS // 2), lambda i: (0, i, 0)),
          pl.BlockSpec((NC, RB, 1), lambda i: (0, i, 0)),
          pl.BlockSpec((1, CLS), lambda i: (0, 0)),
      ],
      out_specs=pl.BlockSpec((RB, CLS), lambda i: (i, 0)),
      out_shape=jax.ShapeDtypeStruct((NP, CLS), f32))(agg2, dip, b2)


def kernel(x, edge_index, W1, b1, W2, b2):
  src3 = edge_index[0].reshape(NS, TCH, K)
  dst3 = edge_index[1].reshape(NS, TCH, K)
  zn = jnp.zeros((NP,), f32)
  z64 = jnp.zeros((NP, HID // 2), f32)
  z32 = jnp.zeros((NP, CLS // 2), f32)
  x_p = jnp.pad(x, ((0, NP - N), (0, 0)))

  dop, dip = _degree_pass(src3, dst3, zn)
  dop = dop.reshape(NC, NP, 1)
  dip = dip.reshape(NC, NP, 1)

  xw1 = _m1(x_p, W1, dop)
  agg1 = _message_pass(xw1, src3, dst3, z64, HID // 2)
  xw2 = _m2(agg1, dop, dip, b1.reshape(1, HID), W2)
  agg2 = _message_pass(xw2, src3, dst3, z32, CLS // 2)
  out = _m3(agg2, dip, b2.reshape(1, CLS))
  return out[:N]


# speedup vs baseline: 12.6422x; 1.2192x over previous
import functools

import jax
import jax.numpy as jnp
from jax import lax
from jax.experimental import pallas as pl
from jax.experimental.pallas import tpu as pltpu
from jax.experimental.pallas import tpu_sc as plsc

N = 10000
NP = 10240
E = 320000
EP = 327680
F = 128
HID = 128
CLS = 64
NC = 2
NS = 16
K = 128
TCH = EP // (NS * K)
DCH = EP // (NC * NS * K)
PH1 = DCH // 2
PH2 = TCH // 2
RPT = NP // NS
RB = 2048
GRID = NP // RB

f32 = jnp.float32

_mesh = plsc.VectorSubcoreMesh(
    core_axis_name="c", subcore_axis_name="s", num_cores=NC, num_subcores=NS)

_sc_untiled = pltpu.CompilerParams(use_tc_tiling_on_sc=False)


def _zero_fill(ref, rows, width):
  @pl.loop(0, rows)
  def _(i):
    for j in range(width // 16):
      ref[i, pl.ds(j * 16, 16)] = jnp.zeros((16,), f32)


def _degree_pass(src2, dst2):
  @functools.partial(
      pl.kernel, mesh=_mesh, compiler_params=_sc_untiled,
      out_type=(jax.ShapeDtypeStruct((NC, NP), f32),
                jax.ShapeDtypeStruct((NC, NP), f32)),
      scratch_types=[
          pltpu.VMEM_SHARED((NP,), f32),
          pltpu.VMEM_SHARED((NP,), f32),
          pltpu.VMEM((DCH, K), jnp.int32),
          pltpu.VMEM((DCH, K), jnp.int32),
          pltpu.VMEM((K,), f32),
          pltpu.VMEM((RPT,), f32),
      ])
  def body(src_h, dst_h, do_h, di_h, do_sh, di_sh, sidx, didx, ones_v, zb):
    c = lax.axis_index("c")
    s = lax.axis_index("s")
    w = c * NS + s
    pltpu.sync_copy(src_h.at[w], sidx)
    pltpu.sync_copy(dst_h.at[w], didx)

    @pl.loop(0, RPT // 16)
    def _(i):
      zb[pl.ds(i * 16, 16)] = jnp.zeros((16,), f32)
    for j in range(K // 16):
      ones_v[pl.ds(j * 16, 16)] = jnp.ones((16,), f32)
    pltpu.sync_copy(zb, do_sh.at[pl.ds(s * RPT, RPT)])
    pltpu.sync_copy(zb, di_sh.at[pl.ds(s * RPT, RPT)])
    plsc.subcore_barrier()

    @pl.loop(0, DCH)
    def _(j):
      pltpu.sync_copy(ones_v, do_sh.at[sidx.at[j]], add=True)
      pltpu.sync_copy(ones_v, di_sh.at[didx.at[j]], add=True)

    plsc.subcore_barrier()
    pltpu.sync_copy(do_sh.at[pl.ds(s * RPT, RPT)], do_h.at[c, pl.ds(s * RPT, RPT)])
    pltpu.sync_copy(di_sh.at[pl.ds(s * RPT, RPT)], di_h.at[c, pl.ds(s * RPT, RPT)])

  return body(src2, dst2)


def _mp1(xw, src2, dst2):
  @functools.partial(
      pl.kernel, mesh=_mesh,
      out_type=jax.ShapeDtypeStruct((NC, NP, F), f32),
      scratch_types=[
          pltpu.VMEM_SHARED((NP, F), f32),
          pltpu.VMEM((PH1, K), jnp.int32),
          pltpu.VMEM((PH1, K), jnp.int32),
          pltpu.VMEM((K, F), f32),
          pltpu.VMEM((K, F), f32),
          pltpu.SemaphoreType.DMA,
          pltpu.SemaphoreType.DMA,
      ])
  def body(xw_h, src_h, dst_h, out_h, acc, sidx, didx, r0, r1, sm0, sm1):
    c = lax.axis_index("c")
    s = lax.axis_index("s")
    w = c * NS + s
    _zero_fill(r0, K, F)
    for k in range(RPT // K):
      pltpu.sync_copy(r0, acc.at[pl.ds(s * RPT + k * K, K)])
    plsc.subcore_barrier()

    for ph in range(2):
      pltpu.sync_copy(src_h.at[w, pl.ds(ph * PH1, PH1)], sidx)
      pltpu.sync_copy(dst_h.at[w, pl.ds(ph * PH1, PH1)], didx)
      pltpu.async_copy(xw_h.at[sidx.at[0]], r0, sm0)

      @pl.loop(0, PH1, step=2)
      def _(g):
        pltpu.async_copy(xw_h.at[sidx.at[g + 1]], r1, sm1)
        pltpu.make_async_copy(xw_h.at[sidx.at[g]], r0, sm0).wait()
        pltpu.sync_copy(r0, acc.at[didx.at[g]], add=True)

        @pl.when(g + 2 < PH1)
        def _():
          pltpu.async_copy(xw_h.at[sidx.at[g + 2]], r0, sm0)

        pltpu.make_async_copy(xw_h.at[sidx.at[g + 1]], r1, sm1).wait()
        pltpu.sync_copy(r1, acc.at[didx.at[g + 1]], add=True)

    plsc.subcore_barrier()
    pltpu.sync_copy(acc.at[pl.ds(s * RPT, RPT)], out_h.at[c, pl.ds(s * RPT, RPT)])

  return body(xw, src2, dst2)


def _mp2(xw, src3, dst3, dh):
  @functools.partial(
      pl.kernel, mesh=_mesh, compiler_params=_sc_untiled,
      out_type=jax.ShapeDtypeStruct((NC, NP, dh), f32),
      scratch_types=[
          pltpu.VMEM_SHARED((NP, dh), f32),
          pltpu.VMEM_SHARED((NP, dh), f32),
          pltpu.VMEM((PH2, K), jnp.int32),
          pltpu.VMEM((PH2, K), jnp.int32),
          pltpu.VMEM((K, dh), f32),
          pltpu.VMEM((K, dh), f32),
          pltpu.SemaphoreType.DMA,
          pltpu.SemaphoreType.DMA,
      ])
  def body(xw_h, src_h, dst_h, out_h, tab, acc, sidx, didx, r0, r1,
           sm0, sm1):
    c = lax.axis_index("c")
    s = lax.axis_index("s")
    pltpu.sync_copy(xw_h.at[c, pl.ds(s * RPT, RPT)], tab.at[pl.ds(s * RPT, RPT)])
    _zero_fill(r0, K, dh)
    for k in range(RPT // K):
      pltpu.sync_copy(r0, acc.at[pl.ds(s * RPT + k * K, K)])
    plsc.subcore_barrier()

    for ph in range(2):
      pltpu.sync_copy(src_h.at[s, pl.ds(ph * PH2, PH2)], sidx)
      pltpu.sync_copy(dst_h.at[s, pl.ds(ph * PH2, PH2)], didx)
      pltpu.async_copy(tab.at[sidx.at[0]], r0, sm0)

      @pl.loop(0, PH2, step=2)
      def _(g):
        pltpu.async_copy(tab.at[sidx.at[g + 1]], r1, sm1)
        pltpu.make_async_copy(tab.at[sidx.at[g]], r0, sm0).wait()
        pltpu.sync_copy(r0, acc.at[didx.at[g]], add=True)

        @pl.when(g + 2 < PH2)
        def _():
          pltpu.async_copy(tab.at[sidx.at[g + 2]], r0, sm0)

        pltpu.make_async_copy(tab.at[sidx.at[g + 1]], r1, sm1).wait()
        pltpu.sync_copy(r1, acc.at[didx.at[g + 1]], add=True)

    plsc.subcore_barrier()
    pltpu.sync_copy(acc.at[pl.ds(s * RPT, RPT)], out_h.at[c, pl.ds(s * RPT, RPT)])

  return body(xw, src3, dst3)


def _norm(dref):
  deg = dref[0] + dref[1]
  return lax.rsqrt(jnp.maximum(deg, 1.0))


def _m1(x_p, W1, dop):
  def body(x_ref, w_ref, d_ref, o_ref):
    xw = jnp.dot(x_ref[...], w_ref[...], preferred_element_type=f32)
    o_ref[...] = xw * _norm(d_ref)

  return pl.pallas_call(
      body, grid=(GRID,),
      in_specs=[
          pl.BlockSpec((RB, F), lambda i: (i, 0)),
          pl.BlockSpec((F, HID), lambda i: (0, 0)),
          pl.BlockSpec((NC, RB, 1), lambda i: (0, i, 0)),
      ],
      out_specs=pl.BlockSpec((RB, HID), lambda i: (i, 0)),
      out_shape=jax.ShapeDtypeStruct((NP, HID), f32))(x_p, W1, dop)


def _m2(agg1, dop, dip, b1, W2):
  def body(a_ref, do_ref, di_ref, b_ref, w_ref, o_ref):
    agg = a_ref[0] + a_ref[1]
    h = jnp.maximum(agg * _norm(di_ref) + b_ref[...], 0.0)
    h = h * _norm(do_ref)
    res = jnp.dot(h, w_ref[...], preferred_element_type=f32)
    o_ref[0] = res[:, :CLS // 2]
    o_ref[1] = res[:, CLS // 2:]

  return pl.pallas_call(
      body, grid=(GRID,),
      in_specs=[
          pl.BlockSpec((NC, RB, HID), lambda i: (0, i, 0)),
          pl.BlockSpec((NC, RB, 1), lambda i: (0, i, 0)),
          pl.BlockSpec((NC, RB, 1), lambda i: (0, i, 0)),
          pl.BlockSpec((1, HID), lambda i: (0, 0)),
          pl.BlockSpec((HID, CLS), lambda i: (0, 0)),
      ],
      out_specs=pl.BlockSpec((NC, RB, CLS // 2), lambda i: (0, i, 0)),
      out_shape=jax.ShapeDtypeStruct((NC, NP, CLS // 2), f32))(
          agg1, dop, dip, b1, W2)


def _m3(agg2, dip, b2):
  def body(a_ref, di_ref, b_ref, o_ref):
    z = jnp.concatenate([a_ref[0], a_ref[1]], axis=1)
    z = z * _norm(di_ref) + b_ref[...]
    z = z - jnp.max(z, axis=1, keepdims=True)
    ez = jnp.exp(z)
    o_ref[...] = ez / jnp.sum(ez, axis=1, keepdims=True)

  return pl.pallas_call(
      body, grid=(GRID,),
      in_specs=[
          pl.BlockSpec((NC, RB, CLS // 2), lambda i: (0, i, 0)),
          pl.BlockSpec((NC, RB, 1), lambda i: (0, i, 0)),
          pl.BlockSpec((1, CLS), lambda i: (0, 0)),
      ],
      out_specs=pl.BlockSpec((RB, CLS), lambda i: (i, 0)),
      out_shape=jax.ShapeDtypeStruct((NP, CLS), f32))(agg2, dip, b2)


def kernel(x, edge_index, W1, b1, W2, b2):
  pad = 10000 + (jnp.arange(EP - E, dtype=jnp.int32) % (NP - N))
  src_p = jnp.concatenate([edge_index[0], pad])
  dst_p = jnp.concatenate([edge_index[1], pad])
  srcA = src_p.reshape(NC * NS, DCH, K)
  dstA = dst_p.reshape(NC * NS, DCH, K)
  srcB = src_p.reshape(NS, TCH, K)
  dstB = dst_p.reshape(NS, TCH, K)
  x_p = jnp.pad(x, ((0, NP - N), (0, 0)))

  dop, dip = _degree_pass(srcA, dstA)
  dop = dop.reshape(NC, NP, 1)
  dip = dip.reshape(NC, NP, 1)

  xw1 = _m1(x_p, W1, dop)
  agg1 = _mp1(xw1, srcA, dstA)
  xw2 = _m2(agg1, dop, dip, b1.reshape(1, HID), W2)
  agg2 = _mp2(xw2, srcB, dstB, CLS // 2)
  out = _m3(agg2, dip, b2.reshape(1, CLS))
  return out[:N]


# speedup vs baseline: 14.1436x; 1.1188x over previous
import functools

import jax
import jax.numpy as jnp
from jax import lax
from jax.experimental import pallas as pl
from jax.experimental.pallas import tpu as pltpu
from jax.experimental.pallas import tpu_sc as plsc

N = 10000
NP = 10240
E = 320000
EP = 327680
F = 128
HID = 128
CLS = 64
NC = 2
NS = 16
K = 128
DCH = EP // (NC * NS * K)
RPT = NP // NS
RB = 2048
GRID = NP // RB

f32 = jnp.float32

_mesh = plsc.VectorSubcoreMesh(
    core_axis_name="c", subcore_axis_name="s", num_cores=NC, num_subcores=NS)

_sc_untiled = pltpu.CompilerParams(use_tc_tiling_on_sc=False)


def _zero_fill(ref, rows, width):
  @pl.loop(0, rows)
  def _(i):
    for j in range(width // 16):
      ref[i, pl.ds(j * 16, 16)] = jnp.zeros((16,), f32)


def _degree_pass(src2, dst2):
  @functools.partial(
      pl.kernel, mesh=_mesh, compiler_params=_sc_untiled,
      out_type=(jax.ShapeDtypeStruct((NC, NP), f32),
                jax.ShapeDtypeStruct((NC, NP), f32)),
      scratch_types=[
          pltpu.VMEM_SHARED((NP,), f32),
          pltpu.VMEM_SHARED((NP,), f32),
          pltpu.VMEM((DCH, K), jnp.int32),
          pltpu.VMEM((DCH, K), jnp.int32),
          pltpu.VMEM((K,), f32),
          pltpu.VMEM((RPT,), f32),
      ])
  def body(src_h, dst_h, do_h, di_h, do_sh, di_sh, sidx, didx, ones_v, zb):
    c = lax.axis_index("c")
    s = lax.axis_index("s")
    w = c * NS + s
    pltpu.sync_copy(src_h.at[w], sidx)
    pltpu.sync_copy(dst_h.at[w], didx)

    @pl.loop(0, RPT // 16)
    def _(i):
      zb[pl.ds(i * 16, 16)] = jnp.zeros((16,), f32)
    for j in range(K // 16):
      ones_v[pl.ds(j * 16, 16)] = jnp.ones((16,), f32)
    pltpu.sync_copy(zb, do_sh.at[pl.ds(s * RPT, RPT)])
    pltpu.sync_copy(zb, di_sh.at[pl.ds(s * RPT, RPT)])
    plsc.subcore_barrier()

    @pl.loop(0, DCH)
    def _(j):
      pltpu.sync_copy(ones_v, do_sh.at[sidx.at[j]], add=True)
      pltpu.sync_copy(ones_v, di_sh.at[didx.at[j]], add=True)

    plsc.subcore_barrier()
    pltpu.sync_copy(do_sh.at[pl.ds(s * RPT, RPT)], do_h.at[c, pl.ds(s * RPT, RPT)])
    pltpu.sync_copy(di_sh.at[pl.ds(s * RPT, RPT)], di_h.at[c, pl.ds(s * RPT, RPT)])

  return body(src2, dst2)


def _mp(xw, src2, dst2, dh, tiled, nph):
  pch = DCH // nph

  @functools.partial(
      pl.kernel, mesh=_mesh,
      compiler_params=None if tiled else _sc_untiled,
      out_type=jax.ShapeDtypeStruct((NC, NP, dh), f32),
      scratch_types=[
          pltpu.VMEM_SHARED((NP, dh), f32),
          pltpu.VMEM((pch, K), jnp.int32),
          pltpu.VMEM((pch, K), jnp.int32),
          pltpu.VMEM((K, dh), f32),
          pltpu.VMEM((K, dh), f32),
          pltpu.SemaphoreType.DMA,
          pltpu.SemaphoreType.DMA,
      ])
  def body(xw_h, src_h, dst_h, out_h, acc, sidx, didx, r0, r1, sm0, sm1):
    c = lax.axis_index("c")
    s = lax.axis_index("s")
    w = c * NS + s
    _zero_fill(r0, K, dh)
    for k in range(RPT // K):
      pltpu.sync_copy(r0, acc.at[pl.ds(s * RPT + k * K, K)])
    plsc.subcore_barrier()

    for ph in range(nph):
      pltpu.sync_copy(src_h.at[w, pl.ds(ph * pch, pch)], sidx)
      pltpu.sync_copy(dst_h.at[w, pl.ds(ph * pch, pch)], didx)
      pltpu.async_copy(xw_h.at[sidx.at[0]], r0, sm0)

      @pl.loop(0, pch, step=2)
      def _(g):
        pltpu.async_copy(xw_h.at[sidx.at[g + 1]], r1, sm1)
        pltpu.make_async_copy(xw_h.at[sidx.at[g]], r0, sm0).wait()
        pltpu.sync_copy(r0, acc.at[didx.at[g]], add=True)

        @pl.when(g + 2 < pch)
        def _():
          pltpu.async_copy(xw_h.at[sidx.at[g + 2]], r0, sm0)

        pltpu.make_async_copy(xw_h.at[sidx.at[g + 1]], r1, sm1).wait()
        pltpu.sync_copy(r1, acc.at[didx.at[g + 1]], add=True)

    plsc.subcore_barrier()
    pltpu.sync_copy(acc.at[pl.ds(s * RPT, RPT)], out_h.at[c, pl.ds(s * RPT, RPT)])

  return body(xw, src2, dst2)


def _norm(dref):
  deg = dref[0] + dref[1]
  return lax.rsqrt(jnp.maximum(deg, 1.0))[:, None]


def _m1(x_p, W1, dop):
  def body(x_ref, w_ref, d_ref, o_ref):
    xw = jnp.dot(x_ref[...], w_ref[...], preferred_element_type=f32)
    o_ref[...] = xw * _norm(d_ref)

  return pl.pallas_call(
      body, grid=(GRID,),
      in_specs=[
          pl.BlockSpec((RB, F), lambda i: (i, 0)),
          pl.BlockSpec((F, HID), lambda i: (0, 0)),
          pl.BlockSpec((NC, RB), lambda i: (0, i)),
      ],
      out_specs=pl.BlockSpec((RB, HID), lambda i: (i, 0)),
      out_shape=jax.ShapeDtypeStruct((NP, HID), f32))(x_p, W1, dop)


def _m2(agg1, dop, dip, b1, W2):
  def body(a_ref, do_ref, di_ref, b_ref, w_ref, o_ref):
    agg = a_ref[0] + a_ref[1]
    h = jnp.maximum(agg * _norm(di_ref) + b_ref[...], 0.0)
    h = h * _norm(do_ref)
    o_ref[...] = jnp.dot(h, w_ref[...], preferred_element_type=f32)

  return pl.pallas_call(
      body, grid=(GRID,),
      in_specs=[
          pl.BlockSpec((NC, RB, HID), lambda i: (0, i, 0)),
          pl.BlockSpec((NC, RB), lambda i: (0, i)),
          pl.BlockSpec((NC, RB), lambda i: (0, i)),
          pl.BlockSpec((1, HID), lambda i: (0, 0)),
          pl.BlockSpec((HID, CLS), lambda i: (0, 0)),
      ],
      out_specs=pl.BlockSpec((RB, CLS), lambda i: (i, 0)),
      out_shape=jax.ShapeDtypeStruct((NP, CLS), f32))(agg1, dop, dip, b1, W2)


def _m3(agg2, dip, b2):
  def body(a_ref, di_ref, b_ref, o_ref):
    z = a_ref[0] + a_ref[1]
    z = z * _norm(di_ref) + b_ref[...]
    z = z - jnp.max(z, axis=1, keepdims=True)
    ez = jnp.exp(z)
    o_ref[...] = ez / jnp.sum(ez, axis=1, keepdims=True)

  return pl.pallas_call(
      body, grid=(GRID,),
      in_specs=[
          pl.BlockSpec((NC, RB, CLS), lambda i: (0, i, 0)),
          pl.BlockSpec((NC, RB), lambda i: (0, i)),
          pl.BlockSpec((1, CLS), lambda i: (0, 0)),
      ],
      out_specs=pl.BlockSpec((RB, CLS), lambda i: (i, 0)),
      out_shape=jax.ShapeDtypeStruct((NP, CLS), f32))(agg2, dip, b2)


def kernel(x, edge_index, W1, b1, W2, b2):
  pad = 10000 + (jnp.arange(EP - E, dtype=jnp.int32) % (NP - N))
  src2 = jnp.concatenate([edge_index[0], pad]).reshape(NC * NS, DCH, K)
  dst2 = jnp.concatenate([edge_index[1], pad]).reshape(NC * NS, DCH, K)

  x_p = jnp.pad(x, ((0, NP - N), (0, 0)))
  dop, dip = _degree_pass(src2, dst2)

  xw1 = _m1(x_p, W1, dop)
  agg1 = _mp(xw1, src2, dst2, HID, True, 2)
  xw2 = _m2(agg1, dop, dip, b1.reshape(1, HID), W2)
  agg2 = _mp(xw2, src2, dst2, CLS, False, 1)
  out = _m3(agg2, dip, b2.reshape(1, CLS))
  return out[:N]
